# baseline (speedup 1.0000x reference)
import functools

import jax
import jax.numpy as jnp
from jax import lax
from jax.experimental import pallas as pl
from jax.experimental.pallas import tpu as pltpu
from jax.experimental.pallas import tpu_sc as plsc

_N = 4096
_D = 32
_R = 256
_NB = _N // _R
_KP1 = 11
_BIG = 1e30
_NCHUNK = 32
_CW = _N // _NCHUNK

_NW = 32
_RPW = _N // _NW
_G = 16
_NG = _RPW // _G
_DEPTH = 128
_PIECE = 512


def _dist_body(src_ref, tgt_ref, wg_ref, bg_ref,
               d_ref, rs_ref, tub_ref, gdd_ref):
    i = pl.program_id(0)

    @pl.when(i == 0)
    def _init():
        mu_s = jnp.mean(src_ref[...], axis=0)
        mu_t = jnp.mean(tgt_ref[...], axis=0)
        diff = mu_s - mu_t
        gdd_ref[0, 0] = jnp.abs(jnp.sum(diff * wg_ref[0, :]) + bg_ref[0])

    x = tgt_ref[...]
    xb = tgt_ref[pl.ds(i * _R, _R), :]
    sq = jnp.sum(x * x, axis=1)
    sqb = jnp.sum(xb * xb, axis=1)
    g = lax.dot_general(xb, x, (((1,), (1,)), ((), ())),
                        preferred_element_type=jnp.float32)
    d2 = sqb[:, None] + sq[None, :] - 2.0 * g
    d2 = jnp.maximum(d2, 0.0)
    dist = jnp.where(d2 > 0.0, jnp.sqrt(jnp.where(d2 > 0.0, d2, 1.0)), 0.0)

    d_ref[...] = dist
    rs_ref[0, :] = jnp.sum(dist, axis=1)

    mins = [jnp.min(dist[:, c * _CW:(c + 1) * _CW], axis=1, keepdims=True)
            for c in range(_NCHUNK)]
    m = jnp.concatenate(mins, axis=1)
    t_prev = jnp.full((_R, 1), -1.0, jnp.float32)
    for _ in range(_KP1):
        masked = jnp.where(m > t_prev, m, _BIG)
        t_prev = jnp.min(masked, axis=1, keepdims=True)
    tub_ref[0, :] = t_prev[:, 0]


def _sc_body(d_hbm, tub_hbm, comp_hbm, slab, buf, tub_v, orow, sem):
    wid = lax.axis_index("s") * 2 + lax.axis_index("c")
    base = wid * _RPW
    lane = lax.broadcasted_iota(jnp.int32, (16,), 0)
    bigv = jnp.full((16,), _BIG, jnp.float32)

    pltpu.sync_copy(tub_hbm.at[pl.ds(base, _RPW)], tub_v)

    def _fill(j, carry):
        buf[pl.ds(j * 16, 16)] = bigv
        return carry
    lax.fori_loop(0, _NG * _DEPTH, _fill, 0)

    cnts = [jnp.zeros((16,), jnp.int32) for _ in range(_NG)]
    for piece in range(_N // _PIECE):
        pltpu.async_copy(
            d_hbm.at[pl.ds(piece * _PIECE, _PIECE), pl.ds(base, _RPW)],
            slab, sem).wait()
        for g in range(_NG):
            tub = tub_v[g * _G:(g + 1) * _G]
            bbase = g * _DEPTH * 16

            def _scan(j, cnt):
                v = slab[j, g * _G:(g + 1) * _G]
                msk = v <= tub
                offs = bbase + cnt + lane
                plsc.store_scatter(buf, [offs], v, mask=msk)
                cnt = cnt + jnp.where(msk, 16, 0).astype(jnp.int32)
                return jnp.minimum(cnt, 16 * (_DEPTH - 1))
            cnts[g] = lax.fori_loop(0, _PIECE, _scan, cnts[g])

    for g in range(_NG):
        bbase = g * _DEPTH * 16
        mdep = jnp.max(cnts[g]) // 16 + 1

        t_prev = jnp.full((16,), -1.0, jnp.float32)
        ts, cs = [], []
        for _ in range(_KP1):
            def _mn(j, t):
                x = buf[pl.ds(bbase + j * 16, 16)]
                return jnp.minimum(t, jnp.where(x > t_prev, x, _BIG))
            tj = lax.fori_loop(0, mdep, _mn, bigv)

            def _ct(j, c):
                x = buf[pl.ds(bbase + j * 16, 16)]
                return c + jnp.where(x <= tj, 1.0, 0.0)
            cj = lax.fori_loop(0, mdep, _ct, jnp.zeros((16,), jnp.float32))
            ts.append(tj)
            cs.append(cj)
            t_prev = tj
        tstar = bigv
        for j in range(_KP1):
            tstar = jnp.minimum(tstar, jnp.where(cs[j] >= _KP1, ts[j], _BIG))

        def _fin(j, sc):
            s, c = sc
            x = buf[pl.ds(bbase + j * 16, 16)]
            lt = x < tstar
            return (s + jnp.where(lt, x, 0.0), c + jnp.where(lt, 1.0, 0.0))
        s_lt, c_lt = lax.fori_loop(
            0, mdep, _fin,
            (jnp.zeros((16,), jnp.float32), jnp.zeros((16,), jnp.float32)))
        s11 = s_lt + tstar * (_KP1 - c_lt)
        orow[...] = (s11 - ts[0]) / 10.0
        pltpu.sync_copy(orow, comp_hbm.at[wid, pl.ds(g * _G, _G)])


def _round_bf16(v):
    y = lax.bitcast_convert_type(v, jnp.int32)
    r = (y + 0x7FFF + ((y >> 16) & 1)) & jnp.int32(-65536)
    return lax.bitcast_convert_type(r, jnp.float32)


def _combine_body(comp_ref, rs_ref, wt_ref, bt_ref, disc_ref):
    comp = comp_ref[...]
    sep = rs_ref[...].reshape(_NW, _RPW) / (_N - 1)
    disc = jnp.abs(_round_bf16(comp) * wt_ref[0, 0]
                   + _round_bf16(sep) * wt_ref[0, 1] + bt_ref[0])
    disc_ref[0, 0] = jnp.sum(disc) / _N


@functools.partial(jax.jit, static_argnames=())
def kernel(source_features, target_features, W_tsdm, b_tsdm, W_gddm, b_gddm):
    d_mat, rs, tub, gdd = pl.pallas_call(
        _dist_body,
        grid=(_NB,),
        in_specs=[
            pl.BlockSpec((_N, _D), lambda i: (0, 0)),
            pl.BlockSpec((_N, _D), lambda i: (0, 0)),
            pl.BlockSpec((1, _D), lambda i: (0, 0)),
            pl.BlockSpec(memory_space=pltpu.SMEM),
        ],
        out_specs=[
            pl.BlockSpec((_R, _N), lambda i: (i, 0)),
            pl.BlockSpec((1, _R), lambda i: (0, i)),
            pl.BlockSpec((1, _R), lambda i: (0, i)),
            pl.BlockSpec(memory_space=pltpu.SMEM),
        ],
        out_shape=[
            jax.ShapeDtypeStruct((_N, _N), jnp.float32),
            jax.ShapeDtypeStruct((1, _N), jnp.float32),
            jax.ShapeDtypeStruct((1, _N), jnp.float32),
            jax.ShapeDtypeStruct((1, 1), jnp.float32),
        ],
        compiler_params=pltpu.CompilerParams(
            dimension_semantics=("arbitrary",),
        ),
    )(source_features, target_features, W_gddm, b_gddm)

    sc_kernel = pl.kernel(
        _sc_body,
        out_type=jax.ShapeDtypeStruct((_NW, _RPW), jnp.float32),
        mesh=plsc.VectorSubcoreMesh(core_axis_name="c", subcore_axis_name="s",
                                    num_cores=2, num_subcores=16),
        compiler_params=pltpu.CompilerParams(needs_layout_passes=False),
        scratch_types=[
            pltpu.VMEM((_PIECE, _RPW), jnp.float32),
            pltpu.VMEM((_NG * _DEPTH * 16,), jnp.float32),
            pltpu.VMEM((_RPW,), jnp.float32),
            pltpu.VMEM((16,), jnp.float32),
            pltpu.SemaphoreType.DMA,
        ],
    )
    comp = sc_kernel(d_mat, tub.reshape(_N))

    wt_b = _round_bf16(W_tsdm)
    disc = pl.pallas_call(
        _combine_body,
        in_specs=[
            pl.BlockSpec((_NW, _RPW), lambda: (0, 0)),
            pl.BlockSpec((1, _N), lambda: (0, 0)),
            pl.BlockSpec(memory_space=pltpu.SMEM),
            pl.BlockSpec(memory_space=pltpu.SMEM),
        ],
        out_specs=pl.BlockSpec(memory_space=pltpu.SMEM),
        out_shape=jax.ShapeDtypeStruct((1, 1), jnp.float32),
    )(comp, rs, wt_b, b_tsdm)

    return (gdd[0, 0], disc[0, 0])


# speedup vs baseline: 1.1235x; 1.1235x over previous
import functools

import jax
import jax.numpy as jnp
from jax import lax
from jax.experimental import pallas as pl
from jax.experimental.pallas import tpu as pltpu
from jax.experimental.pallas import tpu_sc as plsc

_N = 4096
_D = 32
_R = 256
_NB = _N // _R
_KP1 = 11
_BIG = 1e30
_NCHUNK = 32
_CW = _N // _NCHUNK

_NW = 32
_RPW = _N // _NW
_G = 16
_NG = _RPW // _G
_DEPTH = 128
_PIECE = 256


def _dist_body(src_ref, tgt_ref, wg_ref, bg_ref,
               d_ref, rs_ref, tub_ref, gdd_ref):
    i = pl.program_id(0)

    @pl.when(i == 0)
    def _init():
        mu_s = jnp.mean(src_ref[...], axis=0)
        mu_t = jnp.mean(tgt_ref[...], axis=0)
        diff = mu_s - mu_t
        gdd_ref[0, 0] = jnp.abs(jnp.sum(diff * wg_ref[0, :]) + bg_ref[0])

    x = tgt_ref[...]
    xb = tgt_ref[pl.ds(i * _R, _R), :]
    sq = jnp.sum(x * x, axis=1)
    sqb = jnp.sum(xb * xb, axis=1)
    g = lax.dot_general(xb, x, (((1,), (1,)), ((), ())),
                        preferred_element_type=jnp.float32)
    d2 = sqb[:, None] + sq[None, :] - 2.0 * g
    d2 = jnp.maximum(d2, 0.0)
    dist = jnp.where(d2 > 0.0, jnp.sqrt(jnp.where(d2 > 0.0, d2, 1.0)), 0.0)

    d_ref[...] = dist
    rs_ref[0, :] = jnp.sum(dist, axis=1)

    mins = [jnp.min(dist[:, c * _CW:(c + 1) * _CW], axis=1, keepdims=True)
            for c in range(_NCHUNK)]
    m = jnp.concatenate(mins, axis=1)
    t_prev = jnp.full((_R, 1), -1.0, jnp.float32)
    for _ in range(_KP1):
        masked = jnp.where(m > t_prev, m, _BIG)
        t_prev = jnp.min(masked, axis=1, keepdims=True)
    tub_ref[0, :] = t_prev[:, 0]


def _sc_body(d_hbm, tub_hbm, comp_hbm, slab, buf, tub_v, orow, sem0, sem1):
    wid = lax.axis_index("s") * 2 + lax.axis_index("c")
    base = wid * _RPW
    lane = lax.broadcasted_iota(jnp.int32, (16,), 0)
    bigv = jnp.full((16,), _BIG, jnp.float32)

    pltpu.sync_copy(tub_hbm.at[pl.ds(base, _RPW)], tub_v)
    tubs = [tub_v[g * _G:(g + 1) * _G] for g in range(_NG)]
    lanes = [lane + g * _DEPTH * 16 for g in range(_NG)]

    def _fill(j, carry):
        for g in range(_NG):
            buf[pl.ds(g * _DEPTH * 16 + j * 16, 16)] = bigv
        return carry
    lax.fori_loop(0, _DEPTH, _fill, 0)

    npieces = _N // _PIECE
    sems = (sem0, sem1)
    copies = [None, None]
    copies[0] = pltpu.async_copy(
        d_hbm.at[pl.ds(0, _PIECE), pl.ds(base, _RPW)], slab.at[0], sems[0])
    cnts = tuple(jnp.zeros((16,), jnp.int32) for _ in range(_NG))
    for piece in range(npieces):
        cur = piece % 2
        nxt = (piece + 1) % 2
        if piece + 1 < npieces:
            copies[nxt] = pltpu.async_copy(
                d_hbm.at[pl.ds((piece + 1) * _PIECE, _PIECE),
                         pl.ds(base, _RPW)],
                slab.at[nxt], sems[nxt])
        copies[cur].wait()

        def _scan(j, cs):
            out = []
            for g in range(_NG):
                v = slab[cur, j, g * _G:(g + 1) * _G]
                msk = v <= tubs[g]
                offs = cs[g] + lanes[g]
                plsc.store_scatter(buf, [offs], v, mask=msk)
                c = cs[g] + jnp.where(msk, 16, 0).astype(jnp.int32)
                out.append(jnp.minimum(c, 16 * (_DEPTH - 1)))
            return tuple(out)
        cnts = lax.fori_loop(0, _PIECE, _scan, cnts)
    cnts = list(cnts)

    for g in range(_NG):
        bbase = g * _DEPTH * 16
        mdep = jnp.max(cnts[g]) // 16 + 1

        t_prev = jnp.full((16,), -1.0, jnp.float32)
        ts, cs = [], []
        for _ in range(_KP1):
            def _mn(j, t):
                x = buf[pl.ds(bbase + j * 16, 16)]
                return jnp.minimum(t, jnp.where(x > t_prev, x, _BIG))
            tj = lax.fori_loop(0, mdep, _mn, bigv)

            def _ct(j, c):
                x = buf[pl.ds(bbase + j * 16, 16)]
                return c + jnp.where(x <= tj, 1.0, 0.0)
            cj = lax.fori_loop(0, mdep, _ct, jnp.zeros((16,), jnp.float32))
            ts.append(tj)
            cs.append(cj)
            t_prev = tj
        tstar = bigv
        for j in range(_KP1):
            tstar = jnp.minimum(tstar, jnp.where(cs[j] >= _KP1, ts[j], _BIG))

        def _fin(j, sc):
            s, c = sc
            x = buf[pl.ds(bbase + j * 16, 16)]
            lt = x < tstar
            return (s + jnp.where(lt, x, 0.0), c + jnp.where(lt, 1.0, 0.0))
        s_lt, c_lt = lax.fori_loop(
            0, mdep, _fin,
            (jnp.zeros((16,), jnp.float32), jnp.zeros((16,), jnp.float32)))
        s11 = s_lt + tstar * (_KP1 - c_lt)
        orow[...] = (s11 - ts[0]) / 10.0
        pltpu.sync_copy(orow, comp_hbm.at[wid, pl.ds(g * _G, _G)])


def _round_bf16(v):
    y = lax.bitcast_convert_type(v, jnp.int32)
    r = (y + 0x7FFF + ((y >> 16) & 1)) & jnp.int32(-65536)
    return lax.bitcast_convert_type(r, jnp.float32)


def _combine_body(comp_ref, rs_ref, wt_ref, bt_ref, disc_ref):
    comp = comp_ref[...]
    sep = rs_ref[...].reshape(_NW, _RPW) / (_N - 1)
    disc = jnp.abs(_round_bf16(comp) * wt_ref[0, 0]
                   + _round_bf16(sep) * wt_ref[0, 1] + bt_ref[0])
    disc_ref[0, 0] = jnp.sum(disc) / _N


@functools.partial(jax.jit, static_argnames=())
def kernel(source_features, target_features, W_tsdm, b_tsdm, W_gddm, b_gddm):
    d_mat, rs, tub, gdd = pl.pallas_call(
        _dist_body,
        grid=(_NB,),
        in_specs=[
            pl.BlockSpec((_N, _D), lambda i: (0, 0)),
            pl.BlockSpec((_N, _D), lambda i: (0, 0)),
            pl.BlockSpec((1, _D), lambda i: (0, 0)),
            pl.BlockSpec(memory_space=pltpu.SMEM),
        ],
        out_specs=[
            pl.BlockSpec((_R, _N), lambda i: (i, 0)),
            pl.BlockSpec((1, _R), lambda i: (0, i)),
            pl.BlockSpec((1, _R), lambda i: (0, i)),
            pl.BlockSpec(memory_space=pltpu.SMEM),
        ],
        out_shape=[
            jax.ShapeDtypeStruct((_N, _N), jnp.float32),
            jax.ShapeDtypeStruct((1, _N), jnp.float32),
            jax.ShapeDtypeStruct((1, _N), jnp.float32),
            jax.ShapeDtypeStruct((1, 1), jnp.float32),
        ],
        compiler_params=pltpu.CompilerParams(
            dimension_semantics=("arbitrary",),
        ),
    )(source_features, target_features, W_gddm, b_gddm)

    sc_kernel = pl.kernel(
        _sc_body,
        out_type=jax.ShapeDtypeStruct((_NW, _RPW), jnp.float32),
        mesh=plsc.VectorSubcoreMesh(core_axis_name="c", subcore_axis_name="s",
                                    num_cores=2, num_subcores=16),
        compiler_params=pltpu.CompilerParams(needs_layout_passes=False),
        scratch_types=[
            pltpu.VMEM((2, _PIECE, _RPW), jnp.float32),
            pltpu.VMEM((_NG * _DEPTH * 16,), jnp.float32),
            pltpu.VMEM((_RPW,), jnp.float32),
            pltpu.VMEM((16,), jnp.float32),
            pltpu.SemaphoreType.DMA,
            pltpu.SemaphoreType.DMA,
        ],
    )
    comp = sc_kernel(d_mat, tub.reshape(_N))

    wt_b = _round_bf16(W_tsdm)
    disc = pl.pallas_call(
        _combine_body,
        in_specs=[
            pl.BlockSpec((_NW, _RPW), lambda: (0, 0)),
            pl.BlockSpec((1, _N), lambda: (0, 0)),
            pl.BlockSpec(memory_space=pltpu.SMEM),
            pl.BlockSpec(memory_space=pltpu.SMEM),
        ],
        out_specs=pl.BlockSpec(memory_space=pltpu.SMEM),
        out_shape=jax.ShapeDtypeStruct((1, 1), jnp.float32),
    )(comp, rs, wt_b, b_tsdm)

    return (gdd[0, 0], disc[0, 0])
